# baseline (speedup 1.0000x reference)
import functools

import jax
import jax.numpy as jnp
from jax import lax
from jax.experimental import pallas as pl
from jax.experimental.pallas import tpu as pltpu
from jax.experimental.pallas import tpu_sc as plsc

R = 128
V = 100000
L = 16
NB = R // L
NC = 2
NS = 16
NW = NC * NS

BP = 8192
SCV = 57344
SLAB = SCV // NW
NCH = 4
CPOS = SLAB // NCH
PU = 1

TCOFS = SCV // BP
NBLK = (V - SCV + BP - 1) // BP
UT = 8

_NEG_INF = float("-inf")


@functools.partial(
    pl.kernel,
    mesh=plsc.VectorSubcoreMesh(core_axis_name="c", subcore_axis_name="s"),
    out_type=(
        jax.ShapeDtypeStruct((NW, R), jnp.float32),
        jax.ShapeDtypeStruct((NW, R), jnp.int32),
    ),
    scratch_types=[
        pltpu.VMEM((CPOS, R), jnp.float32),
        pltpu.VMEM((CPOS, R), jnp.float32),
        pltpu.VMEM((R,), jnp.float32),
        pltpu.VMEM((R,), jnp.int32),
        pltpu.SemaphoreType.DMA,
        pltpu.SemaphoreType.DMA,
    ],
)
def _argmax_sc(lt_hbm, vals_hbm, idxs_hbm, buf0, buf1, vout, iout,
               sem0, sem1):
    cid = lax.axis_index("c")
    sid = lax.axis_index("s")
    wid = sid * NC + cid
    off = wid * SLAB
    bufs = (buf0, buf1)
    sems = (sem0, sem1)

    def start(c, b):
        pltpu.make_async_copy(
            lt_hbm.at[pl.ds(off + c * CPOS, CPOS), :], bufs[b], sems[b]
        ).start()

    def wait(b):
        pltpu.make_async_copy(
            lt_hbm.at[pl.ds(0, CPOS), :], bufs[b], sems[b]).wait()

    def scan_chunk(c, b, carry):
        base = off + c * CPOS
        buf = bufs[b]

        def body(i, carry):
            ms, mis = carry
            ms, mis = list(ms), list(mis)
            for q in range(PU):
                p = i * PU + q
                it = jnp.full((L,), base + p, jnp.int32)
                for k in range(NB):
                    v = buf[p, pl.ds(k * L, L)]
                    cmp = v > ms[k]
                    ms[k] = jnp.where(cmp, v, ms[k])
                    mis[k] = jnp.where(cmp, it, mis[k])
            return tuple(ms), tuple(mis)

        return lax.fori_loop(0, CPOS // PU, body, carry)

    start(0, 0)
    carry = (
        tuple(jnp.full((L,), _NEG_INF, jnp.float32) for _ in range(NB)),
        tuple(jnp.zeros((L,), jnp.int32) for _ in range(NB)),
    )

    def pair_body(g, flat):
        carry = (flat[:NB], flat[NB:])
        for p in (0, 1):
            c = 2 * g + p

            @pl.when(c + 1 < NCH)
            def _():
                start(c + 1, 1 - p)

            wait(p)
            carry = scan_chunk(c, p, carry)
        return carry[0] + carry[1]

    flat = lax.fori_loop(0, NCH // 2, pair_body, carry[0] + carry[1])
    for k in range(NB):
        vout[pl.ds(k * L, L)] = flat[k]
        iout[pl.ds(k * L, L)] = flat[NB + k]
    pltpu.sync_copy(vout, vals_hbm.at[wid])
    pltpu.sync_copy(iout, idxs_hbm.at[wid])


def _tc_body(lt_ref, vals_ref, idxs_ref, m_ref, mi_ref):
    i = pl.program_id(0)

    @pl.when(i == 0)
    def _():
        m_ref[...] = jnp.full((8 * UT, R), -jnp.inf, jnp.float32)
        mi_ref[...] = jnp.zeros((8 * UT, R), jnp.int32)

    base = (i + TCOFS) * BP
    pos8 = lax.broadcasted_iota(jnp.int32, (8, R), 0)
    ms = [m_ref[pl.ds(8 * k, 8), :] for k in range(UT)]
    mis = [mi_ref[pl.ds(8 * k, 8), :] for k in range(UT)]
    for s in range(BP // 8):
        k = s % UT
        v = lt_ref[pl.ds(s * 8, 8), :]
        pos = pos8 + (base + s * 8)
        cmp = (v > ms[k]) & (pos < V)
        ms[k] = jnp.where(cmp, v, ms[k])
        mis[k] = jnp.where(cmp, pos, mis[k])
    for k in range(UT):
        m_ref[pl.ds(8 * k, 8), :] = ms[k]
        mi_ref[pl.ds(8 * k, 8), :] = mis[k]

    @pl.when(i == NBLK - 1)
    def _():
        vals_ref[...] = m_ref[...]
        idxs_ref[...] = mi_ref[...]


def _tc_argmax(lt):
    return pl.pallas_call(
        _tc_body,
        grid=(NBLK,),
        in_specs=[pl.BlockSpec((BP, R), lambda i: (i + TCOFS, 0))],
        out_specs=(
            pl.BlockSpec((8 * UT, R), lambda i: (0, 0)),
            pl.BlockSpec((8 * UT, R), lambda i: (0, 0)),
        ),
        out_shape=(
            jax.ShapeDtypeStruct((8 * UT, R), jnp.float32),
            jax.ShapeDtypeStruct((8 * UT, R), jnp.int32),
        ),
        scratch_shapes=[
            pltpu.VMEM((8 * UT, R), jnp.float32),
            pltpu.VMEM((8 * UT, R), jnp.int32),
        ],
        compiler_params=pltpu.CompilerParams(
            dimension_semantics=("arbitrary",)),
    )(lt)


def kernel(logits):
    lt = logits.T
    tc_vals, tc_idxs = _tc_argmax(lt)
    sc_vals, sc_idxs = _argmax_sc(lt)
    vals = jnp.concatenate([sc_vals, tc_vals], axis=0)
    idxs = jnp.concatenate([sc_idxs, tc_idxs], axis=0)
    m = jnp.max(vals, axis=0)
    cand = jnp.where(vals == m[None, :], idxs, jnp.int32(V))
    return jnp.min(cand, axis=0)


# speedup vs baseline: 1.0243x; 1.0243x over previous
import functools

import jax
import jax.numpy as jnp
from jax import lax
from jax.experimental import pallas as pl
from jax.experimental.pallas import tpu as pltpu
from jax.experimental.pallas import tpu_sc as plsc

R = 128
V = 100000
L = 16
NB = R // L
NC = 2
NS = 16
NW = NC * NS

BP = 8192
SCV = 24576
SLAB = SCV // NW
NCH = 2
CPOS = SLAB // NCH
PU = 2

TCOFS = SCV // BP
NBLK = (V - SCV + BP - 1) // BP
UT = 8

_NEG_INF = float("-inf")


@functools.partial(
    pl.kernel,
    mesh=plsc.VectorSubcoreMesh(core_axis_name="c", subcore_axis_name="s"),
    out_type=(
        jax.ShapeDtypeStruct((NW, R), jnp.float32),
        jax.ShapeDtypeStruct((NW, R), jnp.int32),
    ),
    scratch_types=[
        pltpu.VMEM((CPOS, R), jnp.float32),
        pltpu.VMEM((CPOS, R), jnp.float32),
        pltpu.VMEM((R,), jnp.float32),
        pltpu.VMEM((R,), jnp.int32),
        pltpu.SemaphoreType.DMA,
        pltpu.SemaphoreType.DMA,
    ],
)
def _argmax_sc(lt_hbm, vals_hbm, idxs_hbm, buf0, buf1, vout, iout,
               sem0, sem1):
    cid = lax.axis_index("c")
    sid = lax.axis_index("s")
    wid = sid * NC + cid
    off = wid * SLAB
    bufs = (buf0, buf1)
    sems = (sem0, sem1)

    def start(c, b):
        pltpu.make_async_copy(
            lt_hbm.at[pl.ds(off + c * CPOS, CPOS), :], bufs[b], sems[b]
        ).start()

    def wait(b):
        pltpu.make_async_copy(
            lt_hbm.at[pl.ds(0, CPOS), :], bufs[b], sems[b]).wait()

    def scan_chunk(c, b, carry):
        base = off + c * CPOS
        buf = bufs[b]

        def body(i, carry):
            ms, mis = carry
            ms, mis = list(ms), list(mis)
            for q in range(PU):
                p = i * PU + q
                it = jnp.full((L,), base + p, jnp.int32)
                for k in range(NB):
                    v = buf[p, pl.ds(k * L, L)]
                    cmp = v > ms[k]
                    ms[k] = jnp.where(cmp, v, ms[k])
                    mis[k] = jnp.where(cmp, it, mis[k])
            return tuple(ms), tuple(mis)

        return lax.fori_loop(0, CPOS // PU, body, carry)

    start(0, 0)
    carry = (
        tuple(jnp.full((L,), _NEG_INF, jnp.float32) for _ in range(NB)),
        tuple(jnp.zeros((L,), jnp.int32) for _ in range(NB)),
    )

    def pair_body(g, flat):
        carry = (flat[:NB], flat[NB:])
        for p in (0, 1):
            c = 2 * g + p

            @pl.when(c + 1 < NCH)
            def _():
                start(c + 1, 1 - p)

            wait(p)
            carry = scan_chunk(c, p, carry)
        return carry[0] + carry[1]

    flat = lax.fori_loop(0, NCH // 2, pair_body, carry[0] + carry[1])
    for k in range(NB):
        vout[pl.ds(k * L, L)] = flat[k]
        iout[pl.ds(k * L, L)] = flat[NB + k]
    pltpu.sync_copy(vout, vals_hbm.at[wid])
    pltpu.sync_copy(iout, idxs_hbm.at[wid])


def _tc_body(lt_ref, vals_ref, idxs_ref, m_ref, mi_ref):
    i = pl.program_id(0)

    @pl.when(i == 0)
    def _():
        m_ref[...] = jnp.full((8 * UT, R), -jnp.inf, jnp.float32)
        mi_ref[...] = jnp.zeros((8 * UT, R), jnp.int32)

    base = (i + TCOFS) * BP
    pos8 = lax.broadcasted_iota(jnp.int32, (8, R), 0)
    ms = [m_ref[pl.ds(8 * k, 8), :] for k in range(UT)]
    mis = [mi_ref[pl.ds(8 * k, 8), :] for k in range(UT)]
    for s in range(BP // 8):
        k = s % UT
        v = lt_ref[pl.ds(s * 8, 8), :]
        pos = pos8 + (base + s * 8)
        cmp = (v > ms[k]) & (pos < V)
        ms[k] = jnp.where(cmp, v, ms[k])
        mis[k] = jnp.where(cmp, pos, mis[k])
    for k in range(UT):
        m_ref[pl.ds(8 * k, 8), :] = ms[k]
        mi_ref[pl.ds(8 * k, 8), :] = mis[k]

    @pl.when(i == NBLK - 1)
    def _():
        vals_ref[...] = m_ref[...]
        idxs_ref[...] = mi_ref[...]


def _tc_argmax(lt):
    return pl.pallas_call(
        _tc_body,
        grid=(NBLK,),
        in_specs=[pl.BlockSpec((BP, R), lambda i: (i + TCOFS, 0))],
        out_specs=(
            pl.BlockSpec((8 * UT, R), lambda i: (0, 0)),
            pl.BlockSpec((8 * UT, R), lambda i: (0, 0)),
        ),
        out_shape=(
            jax.ShapeDtypeStruct((8 * UT, R), jnp.float32),
            jax.ShapeDtypeStruct((8 * UT, R), jnp.int32),
        ),
        scratch_shapes=[
            pltpu.VMEM((8 * UT, R), jnp.float32),
            pltpu.VMEM((8 * UT, R), jnp.int32),
        ],
        compiler_params=pltpu.CompilerParams(
            dimension_semantics=("arbitrary",)),
    )(lt)


def kernel(logits):
    lt = logits.T
    tc_vals, tc_idxs = _tc_argmax(lt)
    sc_vals, sc_idxs = _argmax_sc(lt)
    vals = jnp.concatenate([sc_vals, tc_vals], axis=0)
    idxs = jnp.concatenate([sc_idxs, tc_idxs], axis=0)
    m = jnp.max(vals, axis=0)
    cand = jnp.where(vals == m[None, :], idxs, jnp.int32(V))
    return jnp.min(cand, axis=0)


# speedup vs baseline: 1.0313x; 1.0068x over previous
import functools

import jax
import jax.numpy as jnp
from jax import lax
from jax.experimental import pallas as pl
from jax.experimental.pallas import tpu as pltpu
from jax.experimental.pallas import tpu_sc as plsc

R = 128
V = 100000
L = 16
NB = R // L
NC = 2
NS = 16
NW = NC * NS

BP = 8192
SCV = 49152
SLAB = SCV // NW
NCH = 4
CPOS = SLAB // NCH
PU = 2

TCOFS = SCV // BP
NBLK = (V - SCV + BP - 1) // BP
UT = 8

_NEG_INF = float("-inf")


@functools.partial(
    pl.kernel,
    mesh=plsc.VectorSubcoreMesh(core_axis_name="c", subcore_axis_name="s"),
    out_type=(
        jax.ShapeDtypeStruct((NW, R), jnp.float32),
        jax.ShapeDtypeStruct((NW, R), jnp.int32),
    ),
    scratch_types=[
        pltpu.VMEM((CPOS, R), jnp.float32),
        pltpu.VMEM((CPOS, R), jnp.float32),
        pltpu.VMEM((R,), jnp.float32),
        pltpu.VMEM((R,), jnp.int32),
        pltpu.SemaphoreType.DMA,
        pltpu.SemaphoreType.DMA,
    ],
)
def _argmax_sc(lt_hbm, vals_hbm, idxs_hbm, buf0, buf1, vout, iout,
               sem0, sem1):
    cid = lax.axis_index("c")
    sid = lax.axis_index("s")
    wid = sid * NC + cid
    off = wid * SLAB
    bufs = (buf0, buf1)
    sems = (sem0, sem1)

    def start(c, b):
        pltpu.make_async_copy(
            lt_hbm.at[pl.ds(off + c * CPOS, CPOS), :], bufs[b], sems[b]
        ).start()

    def wait(b):
        pltpu.make_async_copy(
            lt_hbm.at[pl.ds(0, CPOS), :], bufs[b], sems[b]).wait()

    def scan_chunk(c, b, carry):
        base = off + c * CPOS
        buf = bufs[b]

        def body(i, carry):
            ms, mis = carry
            ms, mis = list(ms), list(mis)
            for q in range(PU):
                p = i * PU + q
                it = jnp.full((L,), base + p, jnp.int32)
                for k in range(NB):
                    v = buf[p, pl.ds(k * L, L)]
                    cmp = v > ms[k]
                    ms[k] = jnp.where(cmp, v, ms[k])
                    mis[k] = jnp.where(cmp, it, mis[k])
            return tuple(ms), tuple(mis)

        return lax.fori_loop(0, CPOS // PU, body, carry)

    start(0, 0)
    carry = (
        tuple(jnp.full((L,), _NEG_INF, jnp.float32) for _ in range(NB)),
        tuple(jnp.zeros((L,), jnp.int32) for _ in range(NB)),
    )

    def pair_body(g, flat):
        carry = (flat[:NB], flat[NB:])
        for p in (0, 1):
            c = 2 * g + p

            @pl.when(c + 1 < NCH)
            def _():
                start(c + 1, 1 - p)

            wait(p)
            carry = scan_chunk(c, p, carry)
        return carry[0] + carry[1]

    flat = lax.fori_loop(0, NCH // 2, pair_body, carry[0] + carry[1])
    for k in range(NB):
        vout[pl.ds(k * L, L)] = flat[k]
        iout[pl.ds(k * L, L)] = flat[NB + k]
    pltpu.sync_copy(vout, vals_hbm.at[wid])
    pltpu.sync_copy(iout, idxs_hbm.at[wid])


def _tc_body(lt_ref, vals_ref, idxs_ref, m_ref, mi_ref):
    i = pl.program_id(0)

    @pl.when(i == 0)
    def _():
        m_ref[...] = jnp.full((8 * UT, R), -jnp.inf, jnp.float32)
        mi_ref[...] = jnp.zeros((8 * UT, R), jnp.int32)

    base = (i + TCOFS) * BP
    pos8 = lax.broadcasted_iota(jnp.int32, (8, R), 0)
    ms = [m_ref[pl.ds(8 * k, 8), :] for k in range(UT)]
    mis = [mi_ref[pl.ds(8 * k, 8), :] for k in range(UT)]
    for s in range(BP // 8):
        k = s % UT
        v = lt_ref[pl.ds(s * 8, 8), :]
        pos = pos8 + (base + s * 8)
        cmp = (v > ms[k]) & (pos < V)
        ms[k] = jnp.where(cmp, v, ms[k])
        mis[k] = jnp.where(cmp, pos, mis[k])
    for k in range(UT):
        m_ref[pl.ds(8 * k, 8), :] = ms[k]
        mi_ref[pl.ds(8 * k, 8), :] = mis[k]

    @pl.when(i == NBLK - 1)
    def _():
        vals_ref[...] = m_ref[...]
        idxs_ref[...] = mi_ref[...]


def _tc_argmax(lt):
    return pl.pallas_call(
        _tc_body,
        grid=(NBLK,),
        in_specs=[pl.BlockSpec((BP, R), lambda i: (i + TCOFS, 0))],
        out_specs=(
            pl.BlockSpec((8 * UT, R), lambda i: (0, 0)),
            pl.BlockSpec((8 * UT, R), lambda i: (0, 0)),
        ),
        out_shape=(
            jax.ShapeDtypeStruct((8 * UT, R), jnp.float32),
            jax.ShapeDtypeStruct((8 * UT, R), jnp.int32),
        ),
        scratch_shapes=[
            pltpu.VMEM((8 * UT, R), jnp.float32),
            pltpu.VMEM((8 * UT, R), jnp.int32),
        ],
        compiler_params=pltpu.CompilerParams(
            dimension_semantics=("arbitrary",)),
    )(lt)


def kernel(logits):
    lt = logits.T
    tc_vals, tc_idxs = _tc_argmax(lt)
    sc_vals, sc_idxs = _argmax_sc(lt)
    vals = jnp.concatenate([sc_vals, tc_vals], axis=0)
    idxs = jnp.concatenate([sc_idxs, tc_idxs], axis=0)
    m = jnp.max(vals, axis=0)
    cand = jnp.where(vals == m[None, :], idxs, jnp.int32(V))
    return jnp.min(cand, axis=0)


# speedup vs baseline: 1.0336x; 1.0023x over previous
import functools

import jax
import jax.numpy as jnp
from jax import lax
from jax.experimental import pallas as pl
from jax.experimental.pallas import tpu as pltpu
from jax.experimental.pallas import tpu_sc as plsc

R = 128
V = 100000
L = 16
NB = R // L
NC = 2
NS = 16
NW = NC * NS

BP = 8192
SCV = 49152
SLAB = SCV // NW
NCH = 4
CPOS = SLAB // NCH
PU = 2

TCOFS = SCV // BP
NBLK = (V - SCV + BP - 1) // BP
UT = 8

_NEG_INF = float("-inf")


@functools.partial(
    pl.kernel,
    mesh=plsc.VectorSubcoreMesh(core_axis_name="c", subcore_axis_name="s"),
    out_type=(
        jax.ShapeDtypeStruct((NW, R), jnp.float32),
        jax.ShapeDtypeStruct((NW, R), jnp.int32),
    ),
    scratch_types=[
        pltpu.VMEM((CPOS, R), jnp.float32),
        pltpu.VMEM((CPOS, R), jnp.float32),
        pltpu.VMEM((R,), jnp.float32),
        pltpu.VMEM((R,), jnp.int32),
        pltpu.SemaphoreType.DMA,
        pltpu.SemaphoreType.DMA,
    ],
)
def _argmax_sc(lt_hbm, vals_hbm, idxs_hbm, buf0, buf1, vout, iout,
               sem0, sem1):
    cid = lax.axis_index("c")
    sid = lax.axis_index("s")
    wid = sid * NC + cid
    off = wid * SLAB
    bufs = (buf0, buf1)
    sems = (sem0, sem1)

    def start(c, b):
        pltpu.make_async_copy(
            lt_hbm.at[pl.ds(off + c * CPOS, CPOS), :], bufs[b], sems[b]
        ).start()

    def wait(b):
        pltpu.make_async_copy(
            lt_hbm.at[pl.ds(0, CPOS), :], bufs[b], sems[b]).wait()

    def scan_chunk(c, b, carry):
        base = off + c * CPOS
        buf = bufs[b]

        def body(i, carry):
            ms, mis = carry
            ms, mis = list(ms), list(mis)
            for q in range(PU):
                p = i * PU + q
                it = jnp.full((L,), base + p, jnp.int32)
                for k in range(NB):
                    v = buf[p, pl.ds(k * L, L)]
                    cmp = v > ms[k]
                    ms[k] = jnp.where(cmp, v, ms[k])
                    mis[k] = jnp.where(cmp, it, mis[k])
            return tuple(ms), tuple(mis)

        return lax.fori_loop(0, CPOS // PU, body, carry)

    start(0, 0)
    carry = (
        tuple(jnp.full((L,), _NEG_INF, jnp.float32) for _ in range(NB)),
        tuple(jnp.zeros((L,), jnp.int32) for _ in range(NB)),
    )

    def pair_body(g, flat):
        carry = (flat[:NB], flat[NB:])
        for p in (0, 1):
            c = 2 * g + p

            @pl.when(c + 1 < NCH)
            def _():
                start(c + 1, 1 - p)

            wait(p)
            carry = scan_chunk(c, p, carry)
        return carry[0] + carry[1]

    flat = lax.fori_loop(0, NCH // 2, pair_body, carry[0] + carry[1])
    for k in range(NB):
        vout[pl.ds(k * L, L)] = flat[k]
        iout[pl.ds(k * L, L)] = flat[NB + k]
    pltpu.sync_copy(vout, vals_hbm.at[wid])
    pltpu.sync_copy(iout, idxs_hbm.at[wid])


def _tc_body(lt_ref, vals_ref, idxs_ref, m_ref, mi_ref):
    i = pl.program_id(0)

    @pl.when(i == 0)
    def _():
        m_ref[...] = jnp.full((8 * UT, R), -jnp.inf, jnp.float32)
        mi_ref[...] = jnp.zeros((8 * UT, R), jnp.int32)

    base = (i + TCOFS) * BP
    pos8 = lax.broadcasted_iota(jnp.int32, (8, R), 0)
    ms = [m_ref[pl.ds(8 * k, 8), :] for k in range(UT)]
    mis = [mi_ref[pl.ds(8 * k, 8), :] for k in range(UT)]
    for s in range(BP // 8):
        k = s % UT
        v = lt_ref[pl.ds(s * 8, 8), :]
        pos = pos8 + (base + s * 8)
        cmp = (v > ms[k]) & (pos < V)
        ms[k] = jnp.where(cmp, v, ms[k])
        mis[k] = jnp.where(cmp, pos, mis[k])
    for k in range(UT):
        m_ref[pl.ds(8 * k, 8), :] = ms[k]
        mi_ref[pl.ds(8 * k, 8), :] = mis[k]

    @pl.when(i == NBLK - 1)
    def _():
        m, mi = ms[0], mis[0]
        for k in range(1, UT):
            better = (ms[k] > m) | ((ms[k] == m) & (mis[k] < mi))
            m = jnp.where(better, ms[k], m)
            mi = jnp.where(better, mis[k], mi)
        vals_ref[...] = m
        idxs_ref[...] = mi


def _tc_argmax(lt):
    return pl.pallas_call(
        _tc_body,
        grid=(NBLK,),
        in_specs=[pl.BlockSpec((BP, R), lambda i: (i + TCOFS, 0))],
        out_specs=(
            pl.BlockSpec((8, R), lambda i: (0, 0)),
            pl.BlockSpec((8, R), lambda i: (0, 0)),
        ),
        out_shape=(
            jax.ShapeDtypeStruct((8, R), jnp.float32),
            jax.ShapeDtypeStruct((8, R), jnp.int32),
        ),
        scratch_shapes=[
            pltpu.VMEM((8 * UT, R), jnp.float32),
            pltpu.VMEM((8 * UT, R), jnp.int32),
        ],
        compiler_params=pltpu.CompilerParams(
            dimension_semantics=("arbitrary",)),
    )(lt)


def kernel(logits):
    lt = logits.T
    tc_vals, tc_idxs = _tc_argmax(lt)
    sc_vals, sc_idxs = _argmax_sc(lt)
    vals = jnp.concatenate([sc_vals, tc_vals], axis=0)
    idxs = jnp.concatenate([sc_idxs, tc_idxs], axis=0)
    m = jnp.max(vals, axis=0)
    cand = jnp.where(vals == m[None, :], idxs, jnp.int32(V))
    return jnp.min(cand, axis=0)


# speedup vs baseline: 1.0516x; 1.0174x over previous
import functools

import jax
import jax.numpy as jnp
from jax import lax
from jax.experimental import pallas as pl
from jax.experimental.pallas import tpu as pltpu
from jax.experimental.pallas import tpu_sc as plsc

R = 128
V = 100000
L = 16
NB = R // L
NC = 2
NS = 16
NW = NC * NS

BP = 16384
SCV = 49152
SLAB = SCV // NW
NCH = 4
CPOS = SLAB // NCH
PU = 2

TCOFS = SCV // BP
NBLK = (V - SCV + BP - 1) // BP
UT = 8

_NEG_INF = float("-inf")


@functools.partial(
    pl.kernel,
    mesh=plsc.VectorSubcoreMesh(core_axis_name="c", subcore_axis_name="s"),
    out_type=(
        jax.ShapeDtypeStruct((NW, R), jnp.float32),
        jax.ShapeDtypeStruct((NW, R), jnp.int32),
    ),
    scratch_types=[
        pltpu.VMEM((CPOS, R), jnp.float32),
        pltpu.VMEM((CPOS, R), jnp.float32),
        pltpu.VMEM((R,), jnp.float32),
        pltpu.VMEM((R,), jnp.int32),
        pltpu.SemaphoreType.DMA,
        pltpu.SemaphoreType.DMA,
    ],
)
def _argmax_sc(lt_hbm, vals_hbm, idxs_hbm, buf0, buf1, vout, iout,
               sem0, sem1):
    cid = lax.axis_index("c")
    sid = lax.axis_index("s")
    wid = sid * NC + cid
    off = wid * SLAB
    bufs = (buf0, buf1)
    sems = (sem0, sem1)

    def start(c, b):
        pltpu.make_async_copy(
            lt_hbm.at[pl.ds(off + c * CPOS, CPOS), :], bufs[b], sems[b]
        ).start()

    def wait(b):
        pltpu.make_async_copy(
            lt_hbm.at[pl.ds(0, CPOS), :], bufs[b], sems[b]).wait()

    def scan_chunk(c, b, carry):
        base = off + c * CPOS
        buf = bufs[b]

        def body(i, carry):
            ms, mis = carry
            ms, mis = list(ms), list(mis)
            for q in range(PU):
                p = i * PU + q
                it = jnp.full((L,), base + p, jnp.int32)
                for k in range(NB):
                    v = buf[p, pl.ds(k * L, L)]
                    cmp = v > ms[k]
                    ms[k] = jnp.where(cmp, v, ms[k])
                    mis[k] = jnp.where(cmp, it, mis[k])
            return tuple(ms), tuple(mis)

        return lax.fori_loop(0, CPOS // PU, body, carry)

    start(0, 0)
    carry = (
        tuple(jnp.full((L,), _NEG_INF, jnp.float32) for _ in range(NB)),
        tuple(jnp.zeros((L,), jnp.int32) for _ in range(NB)),
    )

    def pair_body(g, flat):
        carry = (flat[:NB], flat[NB:])
        for p in (0, 1):
            c = 2 * g + p

            @pl.when(c + 1 < NCH)
            def _():
                start(c + 1, 1 - p)

            wait(p)
            carry = scan_chunk(c, p, carry)
        return carry[0] + carry[1]

    flat = lax.fori_loop(0, NCH // 2, pair_body, carry[0] + carry[1])
    for k in range(NB):
        vout[pl.ds(k * L, L)] = flat[k]
        iout[pl.ds(k * L, L)] = flat[NB + k]
    pltpu.sync_copy(vout, vals_hbm.at[wid])
    pltpu.sync_copy(iout, idxs_hbm.at[wid])


def _tc_body(lt_ref, vals_ref, idxs_ref, m_ref, mi_ref):
    i = pl.program_id(0)

    @pl.when(i == 0)
    def _():
        m_ref[...] = jnp.full((8 * UT, R), -jnp.inf, jnp.float32)
        mi_ref[...] = jnp.zeros((8 * UT, R), jnp.int32)

    base = (i + TCOFS) * BP
    pos8 = lax.broadcasted_iota(jnp.int32, (8, R), 0)
    ms = [m_ref[pl.ds(8 * k, 8), :] for k in range(UT)]
    mis = [mi_ref[pl.ds(8 * k, 8), :] for k in range(UT)]
    for s in range(BP // 8):
        k = s % UT
        v = lt_ref[pl.ds(s * 8, 8), :]
        pos = pos8 + (base + s * 8)
        cmp = (v > ms[k]) & (pos < V)
        ms[k] = jnp.where(cmp, v, ms[k])
        mis[k] = jnp.where(cmp, pos, mis[k])
    for k in range(UT):
        m_ref[pl.ds(8 * k, 8), :] = ms[k]
        mi_ref[pl.ds(8 * k, 8), :] = mis[k]

    @pl.when(i == NBLK - 1)
    def _():
        m, mi = ms[0], mis[0]
        for k in range(1, UT):
            better = (ms[k] > m) | ((ms[k] == m) & (mis[k] < mi))
            m = jnp.where(better, ms[k], m)
            mi = jnp.where(better, mis[k], mi)
        vals_ref[...] = m
        idxs_ref[...] = mi


def _tc_argmax(lt):
    return pl.pallas_call(
        _tc_body,
        grid=(NBLK,),
        in_specs=[pl.BlockSpec((BP, R), lambda i: (i + TCOFS, 0))],
        out_specs=(
            pl.BlockSpec((8, R), lambda i: (0, 0)),
            pl.BlockSpec((8, R), lambda i: (0, 0)),
        ),
        out_shape=(
            jax.ShapeDtypeStruct((8, R), jnp.float32),
            jax.ShapeDtypeStruct((8, R), jnp.int32),
        ),
        scratch_shapes=[
            pltpu.VMEM((8 * UT, R), jnp.float32),
            pltpu.VMEM((8 * UT, R), jnp.int32),
        ],
        compiler_params=pltpu.CompilerParams(
            dimension_semantics=("arbitrary",)),
    )(lt)


def kernel(logits):
    lt = logits.T
    tc_vals, tc_idxs = _tc_argmax(lt)
    sc_vals, sc_idxs = _argmax_sc(lt)
    vals = jnp.concatenate([sc_vals, tc_vals], axis=0)
    idxs = jnp.concatenate([sc_idxs, tc_idxs], axis=0)
    m = jnp.max(vals, axis=0)
    cand = jnp.where(vals == m[None, :], idxs, jnp.int32(V))
    return jnp.min(cand, axis=0)
